# baseline (device time: 21848 ns/iter reference)
import jax
import jax.numpy as jnp
from jax import lax
from jax.experimental import pallas as pl
from jax.experimental.pallas import tpu as pltpu

import os

N_DEV = 32
MASKS = tuple(
    int(v) for v in os.environ.get("KMASKS", "1,3,4,8,16").split(",") if v
)
STAGES = len(MASKS)
CHUNKS = int(os.environ.get("KCHUNKS", "4"))


def kernel(x):
    _, m, n = x.shape
    rows = m // CHUNKS

    def body(x_ref, out_ref, acc_ref, comm_ref, send_sems, recv_sems):
        my = lax.axis_index("i")
        peers = [jnp.bitwise_xor(my, mk) for mk in MASKS]

        barrier_sem = pltpu.get_barrier_semaphore()
        for s in range(STAGES):
            pl.semaphore_signal(
                barrier_sem,
                inc=1,
                device_id=(peers[s],),
                device_id_type=pl.DeviceIdType.MESH,
            )
        pl.semaphore_wait(barrier_sem, STAGES)

        for c in range(CHUNKS):
            acc_ref[c, :, :] = x_ref[0, pl.ds(c * rows, rows), :].astype(
                jnp.bfloat16
            )

        def mk_rdma(s, c):
            return pltpu.make_async_remote_copy(
                src_ref=acc_ref.at[c],
                dst_ref=comm_ref.at[s, c],
                send_sem=send_sems.at[s, c],
                recv_sem=recv_sems.at[s, c],
                device_id=(peers[s],),
                device_id_type=pl.DeviceIdType.MESH,
            )

        rdmas = {}
        for c in range(CHUNKS) if STAGES else ():
            rdmas[(0, c)] = mk_rdma(0, c)
            rdmas[(0, c)].start()
        for s in range(STAGES):
            for c in range(CHUNKS):
                rdmas.pop((s, c)).wait()
                acc_ref[c, :, :] += comm_ref[s, c, :, :]
                if s + 1 < STAGES:
                    rdmas[(s + 1, c)] = mk_rdma(s + 1, c)
                    rdmas[(s + 1, c)].start()

        for c in range(CHUNKS):
            out_ref[pl.ds(c * rows, rows), :] = acc_ref[c, :, :].astype(
                jnp.float32
            )

    return pl.pallas_call(
        body,
        out_shape=jax.ShapeDtypeStruct((m, n), jnp.float32),
        in_specs=[pl.BlockSpec(memory_space=pltpu.VMEM)],
        out_specs=pl.BlockSpec(memory_space=pltpu.VMEM),
        scratch_shapes=[
            pltpu.VMEM((CHUNKS, rows, n), jnp.bfloat16),
            pltpu.VMEM((max(STAGES, 1), CHUNKS, rows, n), jnp.bfloat16),
            pltpu.SemaphoreType.DMA((max(STAGES, 1), CHUNKS)),
            pltpu.SemaphoreType.DMA((max(STAGES, 1), CHUNKS)),
        ],
        compiler_params=pltpu.CompilerParams(collective_id=0),
    )(x)


# device time: 18600 ns/iter; 1.1746x vs baseline; 1.1746x over previous
import os

import jax
import jax.numpy as jnp
from jax import lax
from jax.experimental import pallas as pl
from jax.experimental.pallas import tpu as pltpu

N_DEV = 32
Y_MASKS = (3, 4, 7)
Z_MASKS = (8, 16, 24)
CH = int(os.environ.get("KCHUNKS", "4"))


def kernel(x):
    _, m, n = x.shape
    half = m // 2
    rows = half // CH

    def body(
        x_ref,
        out_ref,
        acc_ref,
        comm0_ref,
        commy_ref,
        commz_ref,
        comm3_ref,
        s0_send, s0_recv,
        sy_send, sy_recv,
        sz_send, sz_recv,
        s3_send, s3_recv,
    ):
        my = lax.axis_index("i")
        xpeer = jnp.bitwise_xor(my, 1)
        ypeers = [jnp.bitwise_xor(my, mk) for mk in Y_MASKS]
        zpeers = [jnp.bitwise_xor(my, mk) for mk in Z_MASKS]
        oh = jnp.bitwise_and(jnp.bitwise_xor(my, my >> 1), 1)
        own0 = oh * half
        oth0 = half - own0

        barrier_sem = pltpu.get_barrier_semaphore()
        for peer in [xpeer] + ypeers + zpeers:
            pl.semaphore_signal(
                barrier_sem,
                inc=1,
                device_id=(peer,),
                device_id_type=pl.DeviceIdType.MESH,
            )
        acc_ref[...] = x_ref[0, :, :].astype(jnp.bfloat16)
        pl.semaphore_wait(barrier_sem, 7)

        def exch(src_row, dst_ref, send_sem, recv_sem, peer):
            return pltpu.make_async_remote_copy(
                src_ref=acc_ref.at[pl.ds(src_row, rows)],
                dst_ref=dst_ref,
                send_sem=send_sem,
                recv_sem=recv_sem,
                device_id=(peer,),
                device_id_type=pl.DeviceIdType.MESH,
            )

        r0, r1, r2, r3 = {}, {}, {}, {}
        for c in range(CH):
            r0[c] = exch(
                oth0 + c * rows, comm0_ref.at[c], s0_send.at[c],
                s0_recv.at[c], xpeer,
            )
            r0[c].start()

        for c in range(CH):
            r0[c].wait()
            acc_ref[pl.ds(own0 + c * rows, rows), :] += comm0_ref[c, :, :]
            r1[c] = [
                exch(
                    own0 + c * rows, commy_ref.at[k, c], sy_send.at[k, c],
                    sy_recv.at[k, c], ypeers[k],
                )
                for k in range(3)
            ]
            for r in r1[c]:
                r.start()

        for c in range(CH):
            for r in r1[c]:
                r.wait()
            acc_ref[pl.ds(own0 + c * rows, rows), :] += (
                commy_ref[0, c, :, :]
                + commy_ref[1, c, :, :]
                + commy_ref[2, c, :, :]
            )
            r2[c] = [
                exch(
                    own0 + c * rows, commz_ref.at[k, c], sz_send.at[k, c],
                    sz_recv.at[k, c], zpeers[k],
                )
                for k in range(3)
            ]
            for r in r2[c]:
                r.start()

        for c in range(CH):
            for r in r2[c]:
                r.wait()
            acc_ref[pl.ds(own0 + c * rows, rows), :] += (
                commz_ref[0, c, :, :]
                + commz_ref[1, c, :, :]
                + commz_ref[2, c, :, :]
            )
            r3[c] = exch(
                own0 + c * rows, comm3_ref.at[c], s3_send.at[c],
                s3_recv.at[c], xpeer,
            )
            r3[c].start()
            out_ref[pl.ds(own0 + c * rows, rows), :] = acc_ref[
                pl.ds(own0 + c * rows, rows), :
            ].astype(jnp.float32)

        for c in range(CH):
            r3[c].wait()
            out_ref[pl.ds(oth0 + c * rows, rows), :] = comm3_ref[
                c, :, :
            ].astype(jnp.float32)

    return pl.pallas_call(
        body,
        out_shape=jax.ShapeDtypeStruct((m, n), jnp.float32),
        in_specs=[pl.BlockSpec(memory_space=pltpu.VMEM)],
        out_specs=pl.BlockSpec(memory_space=pltpu.VMEM),
        scratch_shapes=[
            pltpu.VMEM((m, n), jnp.bfloat16),
            pltpu.VMEM((CH, rows, n), jnp.bfloat16),
            pltpu.VMEM((3, CH, rows, n), jnp.bfloat16),
            pltpu.VMEM((3, CH, rows, n), jnp.bfloat16),
            pltpu.VMEM((CH, rows, n), jnp.bfloat16),
            pltpu.SemaphoreType.DMA((CH,)),
            pltpu.SemaphoreType.DMA((CH,)),
            pltpu.SemaphoreType.DMA((3, CH)),
            pltpu.SemaphoreType.DMA((3, CH)),
            pltpu.SemaphoreType.DMA((3, CH)),
            pltpu.SemaphoreType.DMA((3, CH)),
            pltpu.SemaphoreType.DMA((CH,)),
            pltpu.SemaphoreType.DMA((CH,)),
        ],
        compiler_params=pltpu.CompilerParams(collective_id=0),
    )(x)


# device time: 17559 ns/iter; 1.2443x vs baseline; 1.0593x over previous
import os

import jax
import jax.numpy as jnp
from jax import lax
from jax.experimental import pallas as pl
from jax.experimental.pallas import tpu as pltpu

N_DEV = 32
Y_MASKS = (3, 4, 7)
Z_MASKS = (8, 16, 24)
CH = int(os.environ.get("KCHUNKS", "4"))


def kernel(x):
    _, m, n = x.shape
    half = m // 2
    rows = half // CH

    def body(
        x_ref,
        out_ref,
        acc_ref,
        comm0_ref,
        commy_ref,
        commz_ref,
        comm3_ref,
        s0_send, s0_recv,
        sy_send, sy_recv,
        sz_send, sz_recv,
        s3_send, s3_recv,
    ):
        my = lax.axis_index("i")
        xpeer = jnp.bitwise_xor(my, 1)
        ypeers = [jnp.bitwise_xor(my, mk) for mk in Y_MASKS]
        zpeers = [jnp.bitwise_xor(my, mk) for mk in Z_MASKS]
        oh = jnp.bitwise_and(jnp.bitwise_xor(my, my >> 1), 1)
        own0 = oh * half
        oth0 = half - own0

        barrier_sem = pltpu.get_barrier_semaphore()
        for peer in [xpeer] + ypeers + zpeers:
            pl.semaphore_signal(
                barrier_sem,
                inc=1,
                device_id=(peer,),
                device_id_type=pl.DeviceIdType.MESH,
            )
        acc_ref[...] = x_ref[0, :, :].astype(jnp.bfloat16)
        pl.semaphore_wait(barrier_sem, 7)

        def exch(src_row, dst_ref, send_sem, recv_sem, peer):
            return pltpu.make_async_remote_copy(
                src_ref=acc_ref.at[pl.ds(src_row, rows)],
                dst_ref=dst_ref,
                send_sem=send_sem,
                recv_sem=recv_sem,
                device_id=(peer,),
                device_id_type=pl.DeviceIdType.MESH,
            )

        r0, r1, r2, r3 = {}, {}, {}, {}
        for c in range(CH):
            r0[c] = exch(
                oth0 + c * rows, comm0_ref.at[c], s0_send.at[c],
                s0_recv.at[c], xpeer,
            )
            r0[c].start()

        def quad(c, use_y):
            if use_y:
                bufs, ss, sr, prs = commy_ref, sy_send, sy_recv, ypeers
            else:
                bufs, ss, sr, prs = commz_ref, sz_send, sz_recv, zpeers
            rs = [
                exch(
                    own0 + c * rows, bufs.at[k, c], ss.at[k, c],
                    sr.at[k, c], prs[k],
                )
                for k in range(3)
            ]
            add = lambda: (
                bufs[0, c, :, :] + bufs[1, c, :, :] + bufs[2, c, :, :]
            )
            return rs, add

        for c in range(CH):
            r0[c].wait()
            acc_ref[pl.ds(own0 + c * rows, rows), :] += comm0_ref[c, :, :]
            r1[c] = quad(c, use_y=(c % 2 == 0))
            for r in r1[c][0]:
                r.start()

        for c in range(CH):
            for r in r1[c][0]:
                r.wait()
            acc_ref[pl.ds(own0 + c * rows, rows), :] += r1[c][1]()
            r2[c] = quad(c, use_y=(c % 2 == 1))
            for r in r2[c][0]:
                r.start()

        for c in range(CH):
            for r in r2[c][0]:
                r.wait()
            acc_ref[pl.ds(own0 + c * rows, rows), :] += r2[c][1]()
            r3[c] = exch(
                own0 + c * rows, comm3_ref.at[c], s3_send.at[c],
                s3_recv.at[c], xpeer,
            )
            r3[c].start()
            out_ref[pl.ds(own0 + c * rows, rows), :] = acc_ref[
                pl.ds(own0 + c * rows, rows), :
            ].astype(jnp.float32)

        for c in range(CH):
            r3[c].wait()
            out_ref[pl.ds(oth0 + c * rows, rows), :] = comm3_ref[
                c, :, :
            ].astype(jnp.float32)

    return pl.pallas_call(
        body,
        out_shape=jax.ShapeDtypeStruct((m, n), jnp.float32),
        in_specs=[pl.BlockSpec(memory_space=pltpu.VMEM)],
        out_specs=pl.BlockSpec(memory_space=pltpu.VMEM),
        scratch_shapes=[
            pltpu.VMEM((m, n), jnp.bfloat16),
            pltpu.VMEM((CH, rows, n), jnp.bfloat16),
            pltpu.VMEM((3, CH, rows, n), jnp.bfloat16),
            pltpu.VMEM((3, CH, rows, n), jnp.bfloat16),
            pltpu.VMEM((CH, rows, n), jnp.bfloat16),
            pltpu.SemaphoreType.DMA((CH,)),
            pltpu.SemaphoreType.DMA((CH,)),
            pltpu.SemaphoreType.DMA((3, CH)),
            pltpu.SemaphoreType.DMA((3, CH)),
            pltpu.SemaphoreType.DMA((3, CH)),
            pltpu.SemaphoreType.DMA((3, CH)),
            pltpu.SemaphoreType.DMA((CH,)),
            pltpu.SemaphoreType.DMA((CH,)),
        ],
        compiler_params=pltpu.CompilerParams(collective_id=0),
    )(x)
